# Initial kernel scaffold; baseline (speedup 1.0000x reference)
import jax
import jax.numpy as jnp
from jax.experimental import pallas as pl


def kernel(x, edge_index, batch, W1, b1, W2, b2, Wl, bl):
    raise NotImplementedError("write your pallas kernel here")


# speedup vs baseline: 23.5427x; 23.5427x over previous
import functools

import jax
import jax.numpy as jnp
from jax import lax
from jax.experimental import pallas as pl
from jax.experimental.pallas import tpu as pltpu
from jax.experimental.pallas import tpu_sc as plsc

G = 64
BLK = 128
NBUF = 4
NPADROWS = 64
CW = 64


def _mesh():
    return plsc.VectorSubcoreMesh(core_axis_name="c", subcore_axis_name="s",
                                  num_cores=2, num_subcores=16)


def _deg_sc(dst2d, np_nodes, nblk):
    npw = np_nodes // 16

    @functools.partial(
        pl.kernel,
        out_type=(jax.ShapeDtypeStruct((np_nodes, 16), jnp.float32),
                  jax.ShapeDtypeStruct((np_nodes, 16), jnp.float32)),
        mesh=_mesh(),
        scratch_types=[
            pltpu.VMEM((nblk, BLK), jnp.int32),
            pltpu.VMEM((BLK, 16), jnp.float32),
            pltpu.VMEM((BLK, 16), jnp.float32),
            pltpu.VMEM_SHARED((np_nodes, 16), jnp.float32),
        ],
    )
    def k(dst_hbm, out0, out1, didx, ones_v, zero_v, acc):
        c = lax.axis_index("c")
        s = lax.axis_index("s")
        w = s * 2 + c

        @pl.loop(0, BLK)
        def _(i):
            ones_v[i] = jnp.ones((16,), jnp.float32)
            zero_v[i] = jnp.zeros((16,), jnp.float32)

        pltpu.sync_copy(dst_hbm.at[pl.ds(w * nblk, nblk)], didx)
        for t in range(npw // BLK):
            pltpu.sync_copy(zero_v, acc.at[pl.ds(s * npw + t * BLK, BLK)])
        plsc.subcore_barrier()

        @pl.loop(0, nblk)
        def _(j):
            pltpu.sync_copy(ones_v, acc.at[didx.at[j]], add=True)

        plsc.subcore_barrier()

        @pl.when(c == 0)
        def _():
            pltpu.sync_copy(acc.at[pl.ds(s * npw, npw)],
                            out0.at[pl.ds(s * npw, npw)])

        @pl.when(c == 1)
        def _():
            pltpu.sync_copy(acc.at[pl.ds(s * npw, npw)],
                            out1.at[pl.ds(s * npw, npw)])

    return k(dst2d)


@functools.lru_cache(maxsize=None)
def _agg_sc_prog(np_nodes, nblk):
    npw = np_nodes // 16

    @functools.partial(
        pl.kernel,
        out_type=tuple(jax.ShapeDtypeStruct((np_nodes, CW), jnp.float32)
                       for _ in range(4)),
        mesh=_mesh(),
        scratch_types=(
            [pltpu.VMEM((nblk, BLK), jnp.int32)] * 2
            + [pltpu.VMEM((BLK, CW), jnp.float32)] * NBUF
            + [pltpu.VMEM_SHARED((np_nodes, CW), jnp.float32)]
            + [pltpu.SemaphoreType.DMA] * NBUF
        ),
        compiler_params=pltpu.CompilerParams(use_tc_tiling_on_sc=False),
    )
    def k(g0, g1, g2, g3, src_hbm, dst_hbm, o0, o1, o2, o3,
          sidx, didx, r0, r1, r2, r3, acc, m0, m1, m2, m3):
        c = lax.axis_index("c")
        s = lax.axis_index("s")
        rows = (r0, r1, r2, r3)
        sems = (m0, m1, m2, m3)

        pltpu.sync_copy(src_hbm.at[pl.ds(s * nblk, nblk)], sidx)
        pltpu.sync_copy(dst_hbm.at[pl.ds(s * nblk, nblk)], didx)

        def run(g_hbm, out_hbm):
            pltpu.sync_copy(g_hbm.at[pl.ds(s * npw, npw)],
                            acc.at[pl.ds(s * npw, npw)])
            plsc.subcore_barrier()

            for b in range(NBUF):
                pltpu.async_copy(g_hbm.at[sidx.at[b]], rows[b], sems[b])

            @pl.loop(0, nblk, step=NBUF)
            def _(i):
                for b in range(NBUF):
                    j = i + b
                    pltpu.make_async_copy(g_hbm.at[sidx.at[j]],
                                          rows[b], sems[b]).wait()
                    pltpu.sync_copy(rows[b], acc.at[didx.at[j]], add=True)

                    @pl.when(j + NBUF < nblk)
                    def _():
                        pltpu.async_copy(g_hbm.at[sidx.at[j + NBUF]],
                                         rows[b], sems[b])

            plsc.subcore_barrier()
            pltpu.sync_copy(acc.at[pl.ds(s * npw, npw)],
                            out_hbm.at[pl.ds(s * npw, npw)])

        @pl.when(c == 0)
        def _():
            run(g0, o0)
            run(g1, o1)

        @pl.when(c == 1)
        def _():
            run(g2, o2)
            run(g3, o3)

    return k


def _agg_sc(gs, src2d, dst2d, np_nodes, nblk):
    return _agg_sc_prog(np_nodes, nblk)(*gs, src2d, dst2d)


def _mm1_body(x_ref, w_ref, da_ref, db_ref, g0, g1, g2, g3, dis_ref):
    deg = da_ref[:, 0:1] + db_ref[:, 0:1] + 1.0
    dis = lax.rsqrt(deg)
    h = jnp.dot(x_ref[...], w_ref[...], preferred_element_type=jnp.float32)
    g = h * dis
    for t, ref in enumerate((g0, g1, g2, g3)):
        ref[...] = g[:, t * CW:(t + 1) * CW]
    dis_ref[...] = dis


def _mm1(x, w1, dega, degb, nb, nsteps):
    n, f_in = x.shape
    h = w1.shape[1]
    return pl.pallas_call(
        _mm1_body,
        grid=(nsteps,),
        in_specs=[
            pl.BlockSpec((nb, f_in), lambda i: (i, 0)),
            pl.BlockSpec((f_in, h), lambda i: (0, 0)),
            pl.BlockSpec((nb, 16), lambda i: (i, 0)),
            pl.BlockSpec((nb, 16), lambda i: (i, 0)),
        ],
        out_specs=[pl.BlockSpec((nb, CW), lambda i: (i, 0))] * 4
        + [pl.BlockSpec((nb, 1), lambda i: (i, 0))],
        out_shape=[jax.ShapeDtypeStruct((n, CW), jnp.float32)] * 4
        + [jax.ShapeDtypeStruct((n, 1), jnp.float32)],
    )(x, w1, dega, degb)


def _mm2_body(s0, s1, s2, s3, dis_ref, b1_ref, w2_ref, g0, g1, g2, g3):
    dis = dis_ref[...]
    nodes = jnp.concatenate([s0[...], s1[...], s2[...], s3[...]],
                            axis=1) * dis + b1_ref[...]
    hrelu = jnp.maximum(nodes, 0.0)
    h2 = jnp.dot(hrelu, w2_ref[...], preferred_element_type=jnp.float32)
    g2o = h2 * dis
    for t, ref in enumerate((g0, g1, g2, g3)):
        ref[...] = g2o[:, t * CW:(t + 1) * CW]


def _mm2(ss, dis, b1, w2, nb, nsteps):
    n = ss[0].shape[0]
    h = w2.shape[0]
    return pl.pallas_call(
        _mm2_body,
        grid=(nsteps,),
        in_specs=[pl.BlockSpec((nb, CW), lambda i: (i, 0))] * 4
        + [
            pl.BlockSpec((nb, 1), lambda i: (i, 0)),
            pl.BlockSpec((1, h), lambda i: (0, 0)),
            pl.BlockSpec((h, h), lambda i: (0, 0)),
        ],
        out_specs=[pl.BlockSpec((nb, CW), lambda i: (i, 0))] * 4,
        out_shape=[jax.ShapeDtypeStruct((n, CW), jnp.float32)] * 4,
    )(*ss, dis, b1, w2)


def _fin_body(s0, s1, s2, s3, dis_ref, b2_ref, batch_ref, wl_ref, bl_ref,
              out_ref):
    n = s0.shape[0]
    nodes = (jnp.concatenate([s0[...], s1[...], s2[...], s3[...]], axis=1)
             * dis_ref[...] + b2_ref[...])
    gid = lax.broadcasted_iota(jnp.int32, (G, n), 0)
    oh = (gid == batch_ref[...]).astype(jnp.float32)
    sums = jnp.dot(oh, nodes, preferred_element_type=jnp.float32)
    counts = jnp.sum(oh, axis=1, keepdims=True)
    pooled = sums / jnp.maximum(counts, 1.0)
    out_ref[...] = (jnp.dot(pooled, wl_ref[...],
                            preferred_element_type=jnp.float32) + bl_ref[...])


def _fin(ss, dis, b2, batch2d, wl, bl):
    c = wl.shape[1]
    return pl.pallas_call(
        _fin_body,
        out_shape=jax.ShapeDtypeStruct((G, c), jnp.float32),
    )(*ss, dis, b2, batch2d, wl, bl)


def kernel(x, edge_index, batch, W1, b1, W2, b2, Wl, bl):
    n, _ = x.shape
    e = edge_index.shape[1]
    src = edge_index[0]
    dst = edge_index[1]

    npn = ((n + 16 * BLK - 1) // (16 * BLK)) * (16 * BLK)
    xp = jnp.pad(x, ((0, npn - n), (0, 0)))
    batchp = jnp.pad(batch, (0, npn - n), constant_values=G)

    nblk1 = ((e + 32 * BLK - 1) // (32 * BLK) + 7) // 8 * 8
    e1 = 32 * nblk1 * BLK
    padd = n + (jnp.arange(e1 - e, dtype=jnp.int32) % NPADROWS)
    dst1 = jnp.concatenate([dst, padd]).reshape(32 * nblk1, BLK)

    nblk2 = ((e + 16 * BLK - 1) // (16 * BLK) + 7) // 8 * 8
    e2 = 16 * nblk2 * BLK
    pads = jnp.arange(e2 - e, dtype=jnp.int32) % NPADROWS
    src2 = jnp.concatenate([src, pads]).reshape(16 * nblk2, BLK)
    dst2 = jnp.concatenate([dst, n + pads]).reshape(16 * nblk2, BLK)

    dega, degb = _deg_sc(dst1, npn, nblk1)

    nsteps = 5
    nb = npn // nsteps
    *g1s, dis = _mm1(xp, W1, dega, degb, nb, nsteps)
    s1s = _agg_sc(g1s, src2, dst2, npn, nblk2)
    g2s = _mm2(s1s, dis, b1.reshape(1, -1), W2, nb, nsteps)
    s2s = _agg_sc(g2s, src2, dst2, npn, nblk2)
    return _fin(s2s, dis, b2.reshape(1, -1), batchp.reshape(1, npn),
                Wl, bl.reshape(1, -1))
